# Initial kernel scaffold; baseline (speedup 1.0000x reference)
import jax
import jax.numpy as jnp
from jax.experimental import pallas as pl


def kernel(wordid_a, edge_index_a, root_ids_a, wordid_b, edge_index_b, root_ids_b, emb, W_iou, U_iou, b_iou, W_f, U_f, b_f, wh_W, wh_b, wp_W, wp_b, r):
    raise NotImplementedError("write your pallas kernel here")


# speedup vs baseline: 1.1984x; 1.1984x over previous
import functools

import jax
import jax.numpy as jnp
from jax.experimental import pallas as pl
from jax.experimental.pallas import tpu as pltpu

N = 100000
H = 128
NB = 2000


def _init_body(x_ref, wiou_ref, biou_ref, wf_ref, bf_ref, uf_ref,
               ioux_ref, xwf_ref, h_ref, c_ref, huf_ref):
    x = x_ref[...]
    ioux = x @ wiou_ref[...] + biou_ref[...]
    ioux_ref[...] = ioux
    xwf_ref[...] = x @ wf_ref[...] + bf_ref[...]
    i = jax.nn.sigmoid(ioux[:, :H])
    o = jax.nn.sigmoid(ioux[:, H:2 * H])
    u = jnp.tanh(ioux[:, 2 * H:])
    c = i * u
    h = o * jnp.tanh(c)
    c_ref[...] = c
    h_ref[...] = h
    huf_ref[...] = h @ uf_ref[...]


def _node_init(x, W_iou, b_iou, W_f, b_f, U_f):
    n = x.shape[0]
    grid = n // NB
    out = pl.pallas_call(
        _init_body,
        grid=(grid,),
        in_specs=[
            pl.BlockSpec((NB, H), lambda i: (i, 0)),
            pl.BlockSpec((H, 3 * H), lambda i: (0, 0)),
            pl.BlockSpec((3 * H,), lambda i: (0,)),
            pl.BlockSpec((H, H), lambda i: (0, 0)),
            pl.BlockSpec((H,), lambda i: (0,)),
            pl.BlockSpec((H, H), lambda i: (0, 0)),
        ],
        out_specs=[
            pl.BlockSpec((NB, 3 * H), lambda i: (i, 0)),
            pl.BlockSpec((NB, H), lambda i: (i, 0)),
            pl.BlockSpec((NB, H), lambda i: (i, 0)),
            pl.BlockSpec((NB, H), lambda i: (i, 0)),
            pl.BlockSpec((NB, H), lambda i: (i, 0)),
        ],
        out_shape=[
            jax.ShapeDtypeStruct((n, 3 * H), jnp.float32),
            jax.ShapeDtypeStruct((n, H), jnp.float32),
            jax.ShapeDtypeStruct((n, H), jnp.float32),
            jax.ShapeDtypeStruct((n, H), jnp.float32),
            jax.ShapeDtypeStruct((n, H), jnp.float32),
        ],
    )(x, W_iou, b_iou, W_f, b_f, U_f)
    return out


def _round_body(ioux_ref, hsum_ref, csum_ref, uiou_ref, uf_ref,
                h_ref, c_ref, huf_ref, *, last):
    iou = ioux_ref[...] + hsum_ref[...] @ uiou_ref[...]
    i = jax.nn.sigmoid(iou[:, :H])
    o = jax.nn.sigmoid(iou[:, H:2 * H])
    u = jnp.tanh(iou[:, 2 * H:])
    c = i * u + csum_ref[...]
    h = o * jnp.tanh(c)
    c_ref[...] = c
    h_ref[...] = h
    if not last:
        huf_ref[...] = h @ uf_ref[...]


def _round_update(ioux, hsum, csum, U_iou, U_f, last=False):
    n = ioux.shape[0]
    grid = n // NB
    out = pl.pallas_call(
        functools.partial(_round_body, last=last),
        grid=(grid,),
        in_specs=[
            pl.BlockSpec((NB, 3 * H), lambda i: (i, 0)),
            pl.BlockSpec((NB, H), lambda i: (i, 0)),
            pl.BlockSpec((NB, H), lambda i: (i, 0)),
            pl.BlockSpec((H, 3 * H), lambda i: (0, 0)),
            pl.BlockSpec((H, H), lambda i: (0, 0)),
        ],
        out_specs=[
            pl.BlockSpec((NB, H), lambda i: (i, 0)),
            pl.BlockSpec((NB, H), lambda i: (i, 0)),
            pl.BlockSpec((NB, H), lambda i: (i, 0)),
        ],
        out_shape=[
            jax.ShapeDtypeStruct((n, H), jnp.float32),
            jax.ShapeDtypeStruct((n, H), jnp.float32),
            jax.ShapeDtypeStruct((n, H), jnp.float32),
        ],
    )(ioux, hsum, csum, U_iou, U_f)
    return out


def _edge_round(h, c, huf, xwf, src, dst):
    n = h.shape[0]
    hsum = jax.ops.segment_sum(jnp.take(h, src, axis=0), dst, num_segments=n)
    f = jax.nn.sigmoid(jnp.take(xwf, dst, axis=0) + jnp.take(huf, src, axis=0))
    csum = jax.ops.segment_sum(f * jnp.take(c, src, axis=0), dst,
                               num_segments=n)
    return hsum, csum


def _mlp_body(ha_ref, hb_ref, whw_ref, whb_ref, wpw_ref, wpb_ref, r_ref,
              out_ref, pred_ref):
    ha = ha_ref[...]
    hb = hb_ref[...]
    mult = ha * hb
    absd = jnp.abs(ha - hb)
    vec = jnp.concatenate([mult, absd], axis=1)
    hid = jax.nn.sigmoid(vec @ whw_ref[...] + whb_ref[...])
    logits = hid @ wpw_ref[...] + wpb_ref[...]
    out = jax.nn.log_softmax(logits, axis=1)
    out_ref[...] = out
    pred_ref[...] = jnp.exp(out) @ r_ref[...]


def _final_mlp(ha, hb, wh_W, wh_b, wp_W, wp_b, r):
    R = ha.shape[0]
    C = wp_W.shape[1]
    out = pl.pallas_call(
        _mlp_body,
        out_shape=[
            jax.ShapeDtypeStruct((R, C), jnp.float32),
            jax.ShapeDtypeStruct((R,), jnp.float32),
        ],
    )(ha, hb, wh_W, wh_b, wp_W, wp_b, r)
    return out


def _tree(wordid, edge_index, emb, W_iou, U_iou, b_iou, W_f, U_f, b_f):
    x = jnp.take(emb, wordid, axis=0)
    src = edge_index[0]
    dst = edge_index[1]
    ioux, xwf, h, c, huf = _node_init(x, W_iou, b_iou, W_f, b_f, U_f)
    for k in range(3):
        hsum, csum = _edge_round(h, c, huf, xwf, src, dst)
        h, c, huf = _round_update(ioux, hsum, csum, U_iou, U_f, last=(k == 2))
    return h


def kernel(wordid_a, edge_index_a, root_ids_a, wordid_b, edge_index_b,
           root_ids_b, emb, W_iou, U_iou, b_iou, W_f, U_f, b_f, wh_W, wh_b,
           wp_W, wp_b, r):
    h_a = _tree(wordid_a, edge_index_a, emb, W_iou, U_iou, b_iou, W_f, U_f, b_f)
    h_b = _tree(wordid_b, edge_index_b, emb, W_iou, U_iou, b_iou, W_f, U_f, b_f)
    ha = jnp.take(h_a, root_ids_a, axis=0)
    hb = jnp.take(h_b, root_ids_b, axis=0)
    out, pred = _final_mlp(ha, hb, wh_W, wh_b, wp_W, wp_b, r)
    return (out, pred)
